# baseline (speedup 1.0000x reference)
import jax
import jax.numpy as jnp
from jax.experimental import pallas as pl
from jax.experimental.pallas import tpu as pltpu

_BLOCK_ROWS = 512


def _scale_body(x_ref, w_ref, o_ref):
    o_ref[...] = x_ref[...] * w_ref[...]


def kernel(x, weight):
    batch, features = x.shape
    w2 = weight.reshape(1, features)
    grid = (batch // _BLOCK_ROWS,)
    return pl.pallas_call(
        _scale_body,
        grid=grid,
        in_specs=[
            pl.BlockSpec((_BLOCK_ROWS, features), lambda i: (i, 0)),
            pl.BlockSpec((1, features), lambda i: (0, 0)),
        ],
        out_specs=pl.BlockSpec((_BLOCK_ROWS, features), lambda i: (i, 0)),
        out_shape=jax.ShapeDtypeStruct((batch, features), x.dtype),
        compiler_params=pltpu.CompilerParams(
            dimension_semantics=("parallel",),
        ),
        name="fixed_range_scaling",
    )(x, w2)


# speedup vs baseline: 1.0064x; 1.0064x over previous
import jax
import jax.numpy as jnp
from jax.experimental import pallas as pl
from jax.experimental.pallas import tpu as pltpu

_BLOCK_ROWS = 896


def _scale_body(x_ref, w_ref, o_ref):
    o_ref[...] = x_ref[...] * w_ref[...]


def kernel(x, weight):
    batch, features = x.shape
    w2 = weight.reshape(1, features)
    grid = (pl.cdiv(batch, _BLOCK_ROWS),)
    return pl.pallas_call(
        _scale_body,
        grid=grid,
        in_specs=[
            pl.BlockSpec((_BLOCK_ROWS, features), lambda i: (i, 0)),
            pl.BlockSpec((1, features), lambda i: (0, 0)),
        ],
        out_specs=pl.BlockSpec((_BLOCK_ROWS, features), lambda i: (i, 0)),
        out_shape=jax.ShapeDtypeStruct((batch, features), x.dtype),
        compiler_params=pltpu.CompilerParams(
            dimension_semantics=("parallel",),
        ),
        name="fixed_range_scaling",
    )(x, w2)


# speedup vs baseline: 1.0080x; 1.0016x over previous
import jax
import jax.numpy as jnp
from jax.experimental import pallas as pl
from jax.experimental.pallas import tpu as pltpu

_BLOCK_ROWS = 1016


def _scale_body(x_ref, w_ref, o_ref):
    o_ref[...] = x_ref[...] * w_ref[...]


def kernel(x, weight):
    batch, features = x.shape
    w2 = weight.reshape(1, features)
    grid = (pl.cdiv(batch, _BLOCK_ROWS),)
    return pl.pallas_call(
        _scale_body,
        grid=grid,
        in_specs=[
            pl.BlockSpec((_BLOCK_ROWS, features), lambda i: (i, 0)),
            pl.BlockSpec((1, features), lambda i: (0, 0)),
        ],
        out_specs=pl.BlockSpec((_BLOCK_ROWS, features), lambda i: (i, 0)),
        out_shape=jax.ShapeDtypeStruct((batch, features), x.dtype),
        compiler_params=pltpu.CompilerParams(
            dimension_semantics=("parallel",),
            vmem_limit_bytes=67108864,
        ),
        name="fixed_range_scaling",
    )(x, w2)
